# Initial kernel scaffold; baseline (speedup 1.0000x reference)
import jax
import jax.numpy as jnp
from jax.experimental import pallas as pl


def kernel(x, in_degree, out_degree, atom_table, W, b):
    raise NotImplementedError("write your pallas kernel here")


# speedup vs baseline: 3.9930x; 3.9930x over previous
import functools

import jax
import jax.numpy as jnp
from jax import lax
from jax.experimental import pallas as pl
from jax.experimental.pallas import tpu as pltpu
from jax.experimental.pallas import tpu_sc as plsc

_NUM_ATOMS = 100000
_HIDDEN = 64
_B = 4096
_N = 200
_ROWS = _B * _N
_NC, _NS = 2, 16
_NW = _NC * _NS
_RPW = _ROWS // _NW
_CHUNK = 256
_NCH = _RPW // _CHUNK
_GSUB = _CHUNK // 128
_LANES = 16


def _sc_body(x_hbm, din_hbm, dout_hbm, tab_hbm, w_hbm, b_hbm, out_hbm,
             idx_all, s_all, s_tmp, rows0, rows1, wv, bv,
             sem_l, semg0, semg1, semo0, semo1):
    wid = lax.axis_index("s") * _NC + lax.axis_index("c")
    base = wid * _RPW

    pltpu.async_copy(x_hbm.at[pl.ds(base, _RPW)], idx_all, sem_l)
    pltpu.async_copy(din_hbm.at[pl.ds(base, _RPW)], s_all, sem_l)
    pltpu.async_copy(dout_hbm.at[pl.ds(base, _RPW)], s_tmp, sem_l)
    pltpu.sync_copy(w_hbm, wv)
    pltpu.sync_copy(b_hbm, bv)
    pltpu.make_async_copy(x_hbm.at[pl.ds(base, _RPW)], idx_all, sem_l).wait()
    pltpu.make_async_copy(din_hbm.at[pl.ds(base, _RPW)], s_all, sem_l).wait()
    pltpu.make_async_copy(dout_hbm.at[pl.ds(base, _RPW)], s_tmp, sem_l).wait()

    def sum_body(k, c):
        sl = pl.ds(k * _LANES, _LANES)
        s_all[sl] = s_all[sl] + s_tmp[sl]
        return c
    lax.fori_loop(0, _RPW // _LANES, sum_body, 0)

    wg = [wv[pl.ds(g * _LANES, _LANES)] for g in range(_HIDDEN // _LANES)]
    b2 = [bv[pl.ds(g * _LANES, _LANES)] + bv[pl.ds(g * _LANES, _LANES)]
          for g in range(_HIDDEN // _LANES)]

    rows = (rows0, rows1)
    semg = (semg0, semg1)
    semo = (semo0, semo1)

    def gdesc(ci, p, j):
        off = ci * _CHUNK + j * 128
        return pltpu.make_async_copy(
            tab_hbm.at[idx_all.at[pl.ds(off, 128)]],
            rows[p].at[pl.ds(j * 128, 128)],
            semg[p])

    def start_gather(ci, p):
        for j in range(_GSUB):
            gdesc(ci, p, j).start()

    def wait_gather(ci, p):
        for j in range(_GSUB):
            gdesc(ci, p, j).wait()

    def odesc(ci, p):
        return pltpu.make_async_copy(
            rows[p], out_hbm.at[pl.ds(base + ci * _CHUNK, _CHUNK)], semo[p])

    def compute(ci, p):
        rp = rows[p]

        def grp(g, c):
            sv = s_all[pl.ds(ci * _CHUNK + g * _LANES, _LANES)]
            for ii in range(_LANES):
                r_loc = g * _LANES + ii
                bc = lax.gather(
                    sv, jnp.full((_LANES, 1), ii, jnp.int32),
                    lax.GatherDimensionNumbers(
                        offset_dims=(), collapsed_slice_dims=(0,),
                        start_index_map=(0,)),
                    slice_sizes=(1,),
                    mode=lax.GatherScatterMode.PROMISE_IN_BOUNDS)
                for gg in range(_HIDDEN // _LANES):
                    sl = pl.ds(gg * _LANES, _LANES)
                    rp[r_loc, sl] = rp[r_loc, sl] + (bc * wg[gg] + b2[gg])
            return c
        lax.fori_loop(0, _CHUNK // _LANES, grp, 0)

    start_gather(0, 0)

    def trip(t, c):
        i0 = 2 * t

        @pl.when(t > 0)
        def _():
            odesc(i0 - 1, 1).wait()
        start_gather(i0 + 1, 1)
        wait_gather(i0, 0)
        compute(i0, 0)
        odesc(i0, 0).start()

        i1 = 2 * t + 1
        odesc(i1 - 1, 0).wait()

        @pl.when(t < _NCH // 2 - 1)
        def _():
            start_gather(i1 + 1, 0)
        wait_gather(i1, 1)
        compute(i1, 1)
        odesc(i1, 1).start()
        return c

    lax.fori_loop(0, _NCH // 2, trip, 0)
    odesc(_NCH - 1, 1).wait()


_graph_node_feature_sc = functools.partial(
    pl.kernel,
    out_type=jax.ShapeDtypeStruct((_ROWS, _HIDDEN), jnp.float32),
    mesh=plsc.VectorSubcoreMesh(core_axis_name="c", subcore_axis_name="s"),
    compiler_params=pltpu.CompilerParams(use_tc_tiling_on_sc=False),
    scratch_types=[
        pltpu.VMEM((_RPW,), jnp.int32),
        pltpu.VMEM((_RPW,), jnp.float32),
        pltpu.VMEM((_RPW,), jnp.float32),
        pltpu.VMEM((_CHUNK, _HIDDEN), jnp.float32),
        pltpu.VMEM((_CHUNK, _HIDDEN), jnp.float32),
        pltpu.VMEM((_HIDDEN,), jnp.float32),
        pltpu.VMEM((_HIDDEN,), jnp.float32),
        pltpu.SemaphoreType.DMA,
        pltpu.SemaphoreType.DMA,
        pltpu.SemaphoreType.DMA,
        pltpu.SemaphoreType.DMA,
        pltpu.SemaphoreType.DMA,
    ],
)(_sc_body)


def kernel(x, in_degree, out_degree, atom_table, W, b):
    xf = x.reshape(-1).astype(jnp.int32)
    din = in_degree.reshape(-1)
    dout = out_degree.reshape(-1)
    wf = W.reshape(-1)
    out = _graph_node_feature_sc(xf, din, dout, atom_table, wf, b)
    return out.reshape(_B, _N, _HIDDEN)


# speedup vs baseline: 4.0411x; 1.0120x over previous
import functools

import jax
import jax.numpy as jnp
from jax import lax
from jax.experimental import pallas as pl
from jax.experimental.pallas import tpu as pltpu
from jax.experimental.pallas import tpu_sc as plsc

_NUM_ATOMS = 100000
_HIDDEN = 64
_B = 4096
_N = 200
_ROWS = _B * _N
_NC, _NS = 2, 16
_NW = _NC * _NS
_RPW = _ROWS // _NW
_BPW = _B // _NW
_CHUNK = 2 * _N
_NCH = _RPW // _CHUNK
_LANES = 16
_GSEG = ((0, 128), (128, 128), (256, 128), (384, 16))


def _sc_body(x_hbm, din_hbm, dout_hbm, tab_hbm, w_hbm, b_hbm, out_hbm,
             idx_all, s_all, s_tmp, rows0, rows1, wv, bv,
             sem_l, semg0, semg1, semo0, semo1):
    wid = lax.axis_index("s") * _NC + lax.axis_index("c")
    base = wid * _RPW
    bbase = wid * _BPW
    half = _RPW // 2

    pltpu.async_copy(x_hbm.at[pl.ds(base, _RPW)], idx_all, sem_l)
    pltpu.async_copy(din_hbm.at[pl.ds(base, _RPW)],
                     s_all.at[pl.ds(0, _RPW)], sem_l)
    pltpu.async_copy(dout_hbm.at[pl.ds(base, half)], s_tmp, sem_l)
    pltpu.sync_copy(w_hbm, wv)
    pltpu.sync_copy(b_hbm, bv)
    pltpu.make_async_copy(x_hbm.at[pl.ds(base, _RPW)], idx_all, sem_l).wait()
    pltpu.make_async_copy(din_hbm.at[pl.ds(base, _RPW)],
                          s_all.at[pl.ds(0, _RPW)], sem_l).wait()
    pltpu.make_async_copy(dout_hbm.at[pl.ds(base, half)], s_tmp, sem_l).wait()

    def sum_body(k, c):
        sl = pl.ds(k * _LANES, _LANES)
        s_all[sl] = s_all[sl] + s_tmp[sl]
        return c
    lax.fori_loop(0, half // _LANES, sum_body, 0)

    pltpu.async_copy(dout_hbm.at[pl.ds(base + half, half)], s_tmp, sem_l)
    pltpu.make_async_copy(dout_hbm.at[pl.ds(base + half, half)],
                          s_tmp, sem_l).wait()

    def sum_body2(k, c):
        sl = pl.ds(half + k * _LANES, _LANES)
        sl2 = pl.ds(k * _LANES, _LANES)
        s_all[sl] = s_all[sl] + s_tmp[sl2]
        return c
    lax.fori_loop(0, half // _LANES, sum_body2, 0)

    wg = [wv[pl.ds(g * _LANES, _LANES)] for g in range(_HIDDEN // _LANES)]
    b2 = [bv[pl.ds(g * _LANES, _LANES)] + bv[pl.ds(g * _LANES, _LANES)]
          for g in range(_HIDDEN // _LANES)]

    rows = (rows0, rows1)
    semg = (semg0, semg1)
    semo = (semo0, semo1)

    def gdesc(ci, p, seg):
        off, n = seg
        return pltpu.make_async_copy(
            tab_hbm.at[idx_all.at[pl.ds(ci * _CHUNK + off, n)]],
            rows[p].at[pl.ds(off, n)],
            semg[p])

    def start_gather(ci, p):
        for seg in _GSEG:
            gdesc(ci, p, seg).start()

    def wait_gather(ci, p):
        for seg in _GSEG:
            gdesc(ci, p, seg).wait()

    def odescs(ci, p):
        b0 = bbase + 2 * ci
        return (
            pltpu.make_async_copy(rows[p].at[pl.ds(0, _N)],
                                  out_hbm.at[b0], semo[p]),
            pltpu.make_async_copy(rows[p].at[pl.ds(_N, _N)],
                                  out_hbm.at[b0 + 1], semo[p]),
        )

    def start_out(ci, p):
        for d in odescs(ci, p):
            d.start()

    def wait_out(ci, p):
        for d in odescs(ci, p):
            d.wait()

    def compute(ci, p):
        rp = rows[p]

        def grp(g, c):
            sv = s_all[pl.ds(ci * _CHUNK + g * _LANES, _LANES)]
            for ii in range(_LANES):
                r_loc = g * _LANES + ii
                bc = lax.gather(
                    sv, jnp.full((_LANES, 1), ii, jnp.int32),
                    lax.GatherDimensionNumbers(
                        offset_dims=(), collapsed_slice_dims=(0,),
                        start_index_map=(0,)),
                    slice_sizes=(1,),
                    mode=lax.GatherScatterMode.PROMISE_IN_BOUNDS)
                for gg in range(_HIDDEN // _LANES):
                    sl = pl.ds(gg * _LANES, _LANES)
                    rp[r_loc, sl] = rp[r_loc, sl] + (bc * wg[gg] + b2[gg])
            return c
        lax.fori_loop(0, _CHUNK // _LANES, grp, 0)

    start_gather(0, 0)

    def trip(t, c):
        i0 = 2 * t

        @pl.when(t > 0)
        def _():
            wait_out(i0 - 1, 1)
        start_gather(i0 + 1, 1)
        wait_gather(i0, 0)
        compute(i0, 0)
        start_out(i0, 0)

        i1 = 2 * t + 1
        wait_out(i1 - 1, 0)

        @pl.when(t < _NCH // 2 - 1)
        def _():
            start_gather(i1 + 1, 0)
        wait_gather(i1, 1)
        compute(i1, 1)
        start_out(i1, 1)
        return c

    lax.fori_loop(0, _NCH // 2, trip, 0)
    wait_out(_NCH - 1, 1)


_graph_node_feature_sc = functools.partial(
    pl.kernel,
    out_type=jax.ShapeDtypeStruct((_B, _N, _HIDDEN), jnp.float32),
    mesh=plsc.VectorSubcoreMesh(core_axis_name="c", subcore_axis_name="s"),
    compiler_params=pltpu.CompilerParams(use_tc_tiling_on_sc=False),
    scratch_types=[
        pltpu.VMEM((_RPW,), jnp.int32),
        pltpu.VMEM((_RPW + _LANES,), jnp.float32),
        pltpu.VMEM((_RPW // 2,), jnp.float32),
        pltpu.VMEM((_CHUNK, _HIDDEN), jnp.float32),
        pltpu.VMEM((_CHUNK, _HIDDEN), jnp.float32),
        pltpu.VMEM((_HIDDEN,), jnp.float32),
        pltpu.VMEM((_HIDDEN,), jnp.float32),
        pltpu.SemaphoreType.DMA,
        pltpu.SemaphoreType.DMA,
        pltpu.SemaphoreType.DMA,
        pltpu.SemaphoreType.DMA,
        pltpu.SemaphoreType.DMA,
    ],
)(_sc_body)


def kernel(x, in_degree, out_degree, atom_table, W, b):
    xf = x.reshape(-1).astype(jnp.int32)
    din = in_degree.reshape(-1)
    dout = out_degree.reshape(-1)
    wf = W.reshape(-1)
    return _graph_node_feature_sc(xf, din, dout, atom_table, wf, b)


# speedup vs baseline: 4.5303x; 1.1211x over previous
import functools

import jax
import jax.numpy as jnp
from jax import lax
from jax.experimental import pallas as pl
from jax.experimental.pallas import tpu as pltpu
from jax.experimental.pallas import tpu_sc as plsc

_NUM_ATOMS = 100000
_HIDDEN = 64
_B = 4096
_N = 200
_NC, _NS = 2, 16
_NW = _NC * _NS
_BT = _B // 128
_NT = _N // 8
_LANES = 16
_HG = _HIDDEN // _LANES
_CW = 128 * _HIDDEN
_OSTRIDE_N = 8 * _BT * 8 * 128
_OSTRIDE_HT = _BT * 8 * 128


def _sc_body(x4_hbm, din_hbm, dout_hbm, tab_hbm, w_hbm, b_hbm, out_hbm,
             idx_v, s_v, st_v, rows0, rows1, res0, res1, sidx_v, wv, bv,
             stage,
             sem_l, semg0, semg1, semsc0, semsc1, semo0, semo1):
    s_ax = lax.axis_index("s")
    c_ax = lax.axis_index("c")
    j = s_ax * _NC + c_ax
    slot0 = s_ax * (2 * _CW)

    pltpu.async_copy(x4_hbm.at[:, j], idx_v, sem_l)
    pltpu.async_copy(din_hbm.at[:, j], s_v, sem_l)
    pltpu.sync_copy(w_hbm, wv)
    pltpu.sync_copy(b_hbm, bv)
    pltpu.make_async_copy(x4_hbm.at[:, j], idx_v, sem_l).wait()
    pltpu.make_async_copy(din_hbm.at[:, j], s_v, sem_l).wait()

    for q in range(4):
        sl_n = pl.ds(q * 50, 50)
        pltpu.async_copy(dout_hbm.at[sl_n, j], st_v, sem_l)
        pltpu.make_async_copy(dout_hbm.at[sl_n, j], st_v, sem_l).wait()

        def sum_body(k, c, q=q):
            n = q * 50 + k
            for g in range(8):
                sl = pl.ds(g * _LANES, _LANES)
                s_v[n, sl] = s_v[n, sl] + st_v[k, sl]
            return c
        lax.fori_loop(0, 50, sum_body, 0)

    iot = lax.iota(jnp.int32, _LANES)

    def sidx_body(q, c):
        e = q * _LANES + iot
        h = lax.rem(e, _HIDDEN)
        bcol = lax.div(e, _HIDDEN)
        pat = (lax.div(h, 8) * 1024 + lax.rem(h, 8) * 128 + bcol)
        qq = lax.div(q, 8)
        csl = pl.ds(lax.rem(q, 8) * _LANES, _LANES)
        sidx_v[0, qq, csl] = slot0 + pat
        sidx_v[1, qq, csl] = slot0 + _CW + pat
        return c
    lax.fori_loop(0, _CW // _LANES, sidx_body, 0)

    wg = [wv[pl.ds(g * _LANES, _LANES)] for g in range(_HG)]
    b2 = [bv[pl.ds(g * _LANES, _LANES)] + bv[pl.ds(g * _LANES, _LANES)]
          for g in range(_HG)]

    rows = (rows0, rows1)
    res = (res0, res1)
    semg = (semg0, semg1)
    semsc = (semsc0, semsc1)
    semo = (semo0, semo1)

    def gdesc(n, p):
        return pltpu.make_async_copy(
            tab_hbm.at[idx_v.at[n // 8, n % 8]], rows[p], semg[p])

    def scatters(p, start):
        def sc_body(qq, c):
            d = pltpu.make_async_copy(
                res[p].at[pl.ds(qq * 128, 128)],
                stage.at[sidx_v.at[p, qq]],
                semsc[p])
            if start:
                d.start()
            else:
                d.wait()
            return c
        lax.fori_loop(0, _CW // 128, sc_body, 0)

    def odescs(n, p):
        base = slot0 + p * _CW
        ds = []
        for ht in range(8):
            ds.append(pltpu.make_async_copy(
                stage.at[pl.ds(base + ht * 1024, 1024)],
                out_hbm.at[pl.ds(n * _OSTRIDE_N + ht * _OSTRIDE_HT
                                 + j * 1024, 1024)],
                semo[p]))
        return ds

    def compute(n, p):
        rp = rows[p]
        re = res[p]

        def grp(g, c):
            sv = s_v[n, pl.ds(g * _LANES, _LANES)]
            for ii in range(_LANES):
                r_loc = g * _LANES + ii
                bc = lax.gather(
                    sv, jnp.full((_LANES, 1), ii, jnp.int32),
                    lax.GatherDimensionNumbers(
                        offset_dims=(), collapsed_slice_dims=(0,),
                        start_index_map=(0,)),
                    slice_sizes=(1,),
                    mode=lax.GatherScatterMode.PROMISE_IN_BOUNDS)
                for gg in range(_HG):
                    v = (rp[r_loc, pl.ds(gg * _LANES, _LANES)]
                         + (bc * wg[gg] + b2[gg]))
                    re[pl.ds(r_loc * _HIDDEN + gg * _LANES, _LANES)] = v
            return c
        lax.fori_loop(0, 128 // _LANES, grp, 0)

    gdesc(0, 0).start()

    def trip(t, c):
        for p in range(2):
            n = 2 * t + p

            @pl.when(t > 0)
            def _():
                for d in odescs(n - 2, p):
                    d.wait()

            if p == 0:
                gdesc(n + 1, 1).start()
            else:
                @pl.when(t < _N // 2 - 1)
                def _():
                    gdesc(n + 1, 0).start()

            gdesc(n, p).wait()
            compute(n, p)
            scatters(p, True)
            scatters(p, False)
            for d in odescs(n, p):
                d.start()
        return c

    lax.fori_loop(0, _N // 2, trip, 0)
    for d in odescs(_N - 2, 0):
        d.wait()
    for d in odescs(_N - 1, 1):
        d.wait()


_graph_node_feature_sc = functools.partial(
    pl.kernel,
    out_type=jax.ShapeDtypeStruct((_N * 8 * _BT * 8 * 128,), jnp.float32),
    mesh=plsc.VectorSubcoreMesh(core_axis_name="c", subcore_axis_name="s"),
    compiler_params=pltpu.CompilerParams(use_tc_tiling_on_sc=False,
                                         needs_layout_passes=False),
    scratch_types=[
        pltpu.VMEM((_NT, 8, 128), jnp.int32),
        pltpu.VMEM((_N, 128), jnp.float32),
        pltpu.VMEM((50, 128), jnp.float32),
        pltpu.VMEM((128, _HIDDEN), jnp.float32),
        pltpu.VMEM((128, _HIDDEN), jnp.float32),
        pltpu.VMEM((_CW,), jnp.float32),
        pltpu.VMEM((_CW,), jnp.float32),
        pltpu.VMEM((2, _CW // 128, 128), jnp.int32),
        pltpu.VMEM((_HIDDEN,), jnp.float32),
        pltpu.VMEM((_HIDDEN,), jnp.float32),
        pltpu.VMEM_SHARED((_NS * 2 * _CW,), jnp.float32),
        pltpu.SemaphoreType.DMA,
        pltpu.SemaphoreType.DMA,
        pltpu.SemaphoreType.DMA,
        pltpu.SemaphoreType.DMA,
        pltpu.SemaphoreType.DMA,
        pltpu.SemaphoreType.DMA,
        pltpu.SemaphoreType.DMA,
    ],
)(_sc_body)


def kernel(x, in_degree, out_degree, atom_table, W, b):
    x4 = x.astype(jnp.int32).reshape(_BT, 128, _NT, 8).transpose(2, 0, 3, 1)
    din = in_degree.reshape(_BT, 128, _N).transpose(2, 0, 1)
    dout = out_degree.reshape(_BT, 128, _N).transpose(2, 0, 1)
    wf = W.reshape(-1)
    outf = _graph_node_feature_sc(x4, din, dout, atom_table, wf, b)
    out5 = outf.reshape(_N, 8, _BT, 8, 128)
    return out5.transpose(2, 4, 0, 1, 3).reshape(_B, _N, _HIDDEN)


# speedup vs baseline: 5.0798x; 1.1213x over previous
import functools

import jax
import jax.numpy as jnp
from jax import lax
from jax.experimental import pallas as pl
from jax.experimental.pallas import tpu as pltpu
from jax.experimental.pallas import tpu_sc as plsc

_NUM_ATOMS = 100000
_HIDDEN = 64
_B = 4096
_N = 200
_NC, _NS = 2, 16
_NW = _NC * _NS
_BT = _B // 128
_NT = _N // 8
_LANES = 16
_HG = _HIDDEN // _LANES
_CW = 128 * _HIDDEN
_OSTRIDE_N = 8 * _BT * 8 * 128
_OSTRIDE_HT = _BT * 8 * 128


def _sc_body(x4_hbm, din_hbm, dout_hbm, tab_hbm, w_hbm, b_hbm, out_hbm,
             idx_v, s_v, st_v, rows0, rows1, res0, res1, sidx_v, wv, bv,
             stage,
             sem_l, semg0, semg1, semsc0, semsc1, semo0, semo1):
    s_ax = lax.axis_index("s")
    c_ax = lax.axis_index("c")
    j = s_ax * _NC + c_ax
    slot0 = s_ax * (3 * _CW)

    pltpu.async_copy(x4_hbm.at[:, j], idx_v, sem_l)
    pltpu.async_copy(din_hbm.at[:, j], s_v, sem_l)
    pltpu.sync_copy(w_hbm, wv)
    pltpu.sync_copy(b_hbm, bv)
    pltpu.make_async_copy(x4_hbm.at[:, j], idx_v, sem_l).wait()
    pltpu.make_async_copy(din_hbm.at[:, j], s_v, sem_l).wait()

    for q in range(4):
        sl_n = pl.ds(q * 50, 50)
        pltpu.async_copy(dout_hbm.at[sl_n, j], st_v, sem_l)
        pltpu.make_async_copy(dout_hbm.at[sl_n, j], st_v, sem_l).wait()

        def sum_body(k, c, q=q):
            n = q * 50 + k
            for g in range(8):
                sl = pl.ds(g * _LANES, _LANES)
                s_v[n, sl] = s_v[n, sl] + st_v[k, sl]
            return c
        lax.fori_loop(0, 50, sum_body, 0)

    iot = lax.iota(jnp.int32, _LANES)

    def sidx_body(q, c):
        e = q * _LANES + iot
        h = lax.rem(e, _HIDDEN)
        bcol = lax.div(e, _HIDDEN)
        pat = (lax.div(h, 8) * 1024 + lax.rem(h, 8) * 128 + bcol)
        qq = lax.div(q, 8)
        csl = pl.ds(lax.rem(q, 8) * _LANES, _LANES)
        sidx_v[qq, csl] = pat
        return c
    lax.fori_loop(0, _CW // _LANES, sidx_body, 0)

    wg = [wv[pl.ds(g * _LANES, _LANES)] for g in range(_HG)]
    b2 = [bv[pl.ds(g * _LANES, _LANES)] + bv[pl.ds(g * _LANES, _LANES)]
          for g in range(_HG)]

    rows = (rows0, rows1)
    res = (res0, res1)
    semg = (semg0, semg1)
    semsc = (semsc0, semsc1)
    semo = (semo0, semo1)

    def gdesc(n, p):
        return pltpu.make_async_copy(
            tab_hbm.at[idx_v.at[n // 8, n % 8]], rows[p], semg[p])

    def scatters(p, s, start):
        def sc_body(qq, c):
            d = pltpu.make_async_copy(
                res[p].at[pl.ds(qq * 128, 128)],
                stage.at[pl.ds(slot0 + s * _CW, _CW)].at[sidx_v.at[qq]],
                semsc[p])
            if start:
                d.start()
            else:
                d.wait()
            return c
        lax.fori_loop(0, _CW // 128, sc_body, 0)

    def odescs(n, p, s):
        base = slot0 + s * _CW
        ds = []
        for ht in range(8):
            ds.append(pltpu.make_async_copy(
                stage.at[pl.ds(base + ht * 1024, 1024)],
                out_hbm.at[pl.ds(n * _OSTRIDE_N + ht * _OSTRIDE_HT
                                 + j * 1024, 1024)],
                semo[p]))
        return ds

    def compute(n, p):
        rp = rows[p]
        re = res[p]

        def grp(g, c):
            sv = s_v[n, pl.ds(g * _LANES, _LANES)]
            for ii in range(_LANES):
                r_loc = g * _LANES + ii
                bc = lax.gather(
                    sv, jnp.full((_LANES, 1), ii, jnp.int32),
                    lax.GatherDimensionNumbers(
                        offset_dims=(), collapsed_slice_dims=(0,),
                        start_index_map=(0,)),
                    slice_sizes=(1,),
                    mode=lax.GatherScatterMode.PROMISE_IN_BOUNDS)
                for gg in range(_HG):
                    v = (rp[r_loc, pl.ds(gg * _LANES, _LANES)]
                         + (bc * wg[gg] + b2[gg]))
                    re[pl.ds(r_loc * _HIDDEN + gg * _LANES, _LANES)] = v
            return c
        lax.fori_loop(0, 128 // _LANES, grp, 0)

    gdesc(0, 0).start()

    def trip(t, c):
        for p in range(2):
            n = 2 * t + p
            s = n % 3

            @pl.when(t > 1 if p == 0 else t > 0)
            def _():
                for d in odescs(n - 3, 1 - p, s):
                    d.wait()

            @pl.when(t > 0)
            def _():
                scatters(p, (n - 2) % 3, False)
                for d in odescs(n - 2, p, (n - 2) % 3):
                    d.start()

            if p == 0:
                gdesc(n + 1, 1).start()
            else:
                @pl.when(t < _N // 2 - 1)
                def _():
                    gdesc(n + 1, 0).start()

            gdesc(n, p).wait()
            compute(n, p)
            scatters(p, s, True)
        return c

    lax.fori_loop(0, _N // 2, trip, 0)
    scatters(0, (_N - 2) % 3, False)
    for d in odescs(_N - 2, 0, (_N - 2) % 3):
        d.start()
    scatters(1, (_N - 1) % 3, False)
    for d in odescs(_N - 1, 1, (_N - 1) % 3):
        d.start()
    for d in odescs(_N - 3, 1, (_N - 3) % 3):
        d.wait()
    for d in odescs(_N - 2, 0, (_N - 2) % 3):
        d.wait()
    for d in odescs(_N - 1, 1, (_N - 1) % 3):
        d.wait()


_graph_node_feature_sc = functools.partial(
    pl.kernel,
    out_type=jax.ShapeDtypeStruct((_N * 8 * _BT * 8 * 128,), jnp.float32),
    mesh=plsc.VectorSubcoreMesh(core_axis_name="c", subcore_axis_name="s"),
    compiler_params=pltpu.CompilerParams(use_tc_tiling_on_sc=False,
                                         needs_layout_passes=False),
    scratch_types=[
        pltpu.VMEM((_NT, 8, 128), jnp.int32),
        pltpu.VMEM((_N, 128), jnp.float32),
        pltpu.VMEM((50, 128), jnp.float32),
        pltpu.VMEM((128, _HIDDEN), jnp.float32),
        pltpu.VMEM((128, _HIDDEN), jnp.float32),
        pltpu.VMEM((_CW,), jnp.float32),
        pltpu.VMEM((_CW,), jnp.float32),
        pltpu.VMEM((_CW // 128, 128), jnp.int32),
        pltpu.VMEM((_HIDDEN,), jnp.float32),
        pltpu.VMEM((_HIDDEN,), jnp.float32),
        pltpu.VMEM_SHARED((_NS * 3 * _CW,), jnp.float32),
        pltpu.SemaphoreType.DMA,
        pltpu.SemaphoreType.DMA,
        pltpu.SemaphoreType.DMA,
        pltpu.SemaphoreType.DMA,
        pltpu.SemaphoreType.DMA,
        pltpu.SemaphoreType.DMA,
        pltpu.SemaphoreType.DMA,
    ],
)(_sc_body)


def kernel(x, in_degree, out_degree, atom_table, W, b):
    x4 = x.astype(jnp.int32).reshape(_BT, 128, _NT, 8).transpose(2, 0, 3, 1)
    din = in_degree.reshape(_BT, 128, _N).transpose(2, 0, 1)
    dout = out_degree.reshape(_BT, 128, _N).transpose(2, 0, 1)
    wf = W.reshape(-1)
    outf = _graph_node_feature_sc(x4, din, dout, atom_table, wf, b)
    out5 = outf.reshape(_N, 8, _BT, 8, 128)
    return out5.transpose(2, 4, 0, 1, 3).reshape(_B, _N, _HIDDEN)


# speedup vs baseline: 5.1200x; 1.0079x over previous
import functools

import jax
import jax.numpy as jnp
from jax import lax
from jax.experimental import pallas as pl
from jax.experimental.pallas import tpu as pltpu
from jax.experimental.pallas import tpu_sc as plsc

_NUM_ATOMS = 100000
_HIDDEN = 64
_B = 4096
_N = 200
_NC, _NS = 2, 16
_NW = _NC * _NS
_BT = _B // 128
_NT = _N // 8
_LANES = 16
_HG = _HIDDEN // _LANES
_CW = 128 * _HIDDEN
_OSTRIDE_N = 8 * _BT * 8 * 128
_OSTRIDE_HT = _BT * 8 * 128


def _sc_body(x4_hbm, din_hbm, dout_hbm, tab_hbm, w_hbm, b_hbm, out_hbm,
             idx_v, s_v, st_v, rows0, rows1, res0, res1, sidx_v, wv, bv,
             stage,
             sem_l, semg0, semg1, semsc0, semsc1, semo0, semo1):
    s_ax = lax.axis_index("s")
    c_ax = lax.axis_index("c")
    j = s_ax * _NC + c_ax
    slot0 = s_ax * (3 * _CW)

    pltpu.async_copy(x4_hbm.at[:, j], idx_v, sem_l)
    pltpu.async_copy(din_hbm.at[:, j], s_v, sem_l)
    pltpu.sync_copy(w_hbm, wv)
    pltpu.sync_copy(b_hbm, bv)
    pltpu.make_async_copy(x4_hbm.at[:, j], idx_v, sem_l).wait()
    pltpu.make_async_copy(din_hbm.at[:, j], s_v, sem_l).wait()

    for q in range(4):
        sl_n = pl.ds(q * 50, 50)
        pltpu.async_copy(dout_hbm.at[sl_n, j], st_v, sem_l)
        pltpu.make_async_copy(dout_hbm.at[sl_n, j], st_v, sem_l).wait()

        def sum_body(k, c, q=q):
            n = q * 50 + k
            for g in range(8):
                sl = pl.ds(g * _LANES, _LANES)
                s_v[n, sl] = s_v[n, sl] + st_v[k, sl]
            return c
        lax.fori_loop(0, 50, sum_body, 0)

    iot = lax.iota(jnp.int32, _LANES)

    def sidx_body(q, c):
        e = q * _LANES + iot
        h = lax.rem(e, _HIDDEN)
        bcol = lax.div(e, _HIDDEN)
        pat = (lax.div(h, 8) * 1024 + lax.rem(h, 8) * 128 + bcol)
        qq = lax.div(q, 8)
        csl = pl.ds(lax.rem(q, 8) * _LANES, _LANES)
        sidx_v[qq, csl] = pat
        return c
    lax.fori_loop(0, _CW // _LANES, sidx_body, 0)

    wg = [wv[pl.ds(g * _LANES, _LANES)] for g in range(_HG)]
    b2 = [bv[pl.ds(g * _LANES, _LANES)] + bv[pl.ds(g * _LANES, _LANES)]
          for g in range(_HG)]

    rows = (rows0, rows1)
    res = (res0, res1)
    semg = (semg0, semg1)
    semsc = (semsc0, semsc1)
    semo = (semo0, semo1)

    def gdesc(n, p):
        return pltpu.make_async_copy(
            tab_hbm.at[idx_v.at[n // 8, n % 8]], rows[p], semg[p])

    def scatters(p, s, start):
        if start:
            def sc_body(qq, c):
                pltpu.make_async_copy(
                    res[p].at[pl.ds(qq * 128, 128)],
                    stage.at[pl.ds(slot0 + s * _CW, _CW)].at[sidx_v.at[qq]],
                    semsc[p]).start()
                return c
            lax.fori_loop(0, _CW // 128, sc_body, 0)
        else:
            pltpu.make_async_copy(
                res[p], stage.at[pl.ds(slot0 + s * _CW, _CW)],
                semsc[p]).wait()

    def ostart(n, p, s):
        base = slot0 + s * _CW
        for ht in range(8):
            pltpu.make_async_copy(
                stage.at[pl.ds(base + ht * 1024, 1024)],
                out_hbm.at[pl.ds(n * _OSTRIDE_N + ht * _OSTRIDE_HT
                                 + j * 1024, 1024)],
                semo[p]).start()

    def owait(p, s):
        pltpu.make_async_copy(
            stage.at[pl.ds(slot0 + s * _CW, _CW)],
            out_hbm.at[pl.ds(j * 1024 * 8, _CW)], semo[p]).wait()

    def compute(n, p):
        rp = rows[p]
        re = res[p]

        def grp(g, c):
            sv = s_v[n, pl.ds(g * _LANES, _LANES)]
            for ii in range(_LANES):
                r_loc = g * _LANES + ii
                bc = lax.gather(
                    sv, jnp.full((_LANES, 1), ii, jnp.int32),
                    lax.GatherDimensionNumbers(
                        offset_dims=(), collapsed_slice_dims=(0,),
                        start_index_map=(0,)),
                    slice_sizes=(1,),
                    mode=lax.GatherScatterMode.PROMISE_IN_BOUNDS)
                for gg in range(_HG):
                    v = (rp[r_loc, pl.ds(gg * _LANES, _LANES)]
                         + (bc * wg[gg] + b2[gg]))
                    re[pl.ds(r_loc * _HIDDEN + gg * _LANES, _LANES)] = v
            return c
        lax.fori_loop(0, 128 // _LANES, grp, 0)

    gdesc(0, 0).start()

    def trip(t, c):
        for p in range(2):
            n = 2 * t + p
            s = n % 3

            @pl.when(t > 1 if p == 0 else t > 0)
            def _():
                owait(1 - p, s)

            @pl.when(t > 0)
            def _():
                scatters(p, (n - 2) % 3, False)
                ostart(n - 2, p, (n - 2) % 3)

            if p == 0:
                gdesc(n + 1, 1).start()
            else:
                @pl.when(t < _N // 2 - 1)
                def _():
                    gdesc(n + 1, 0).start()

            gdesc(n, p).wait()
            compute(n, p)
            scatters(p, s, True)
        return c

    lax.fori_loop(0, _N // 2, trip, 0)
    scatters(0, (_N - 2) % 3, False)
    ostart(_N - 2, 0, (_N - 2) % 3)
    scatters(1, (_N - 1) % 3, False)
    ostart(_N - 1, 1, (_N - 1) % 3)
    owait(1, (_N - 3) % 3)
    owait(0, (_N - 2) % 3)
    owait(1, (_N - 1) % 3)


_graph_node_feature_sc = functools.partial(
    pl.kernel,
    out_type=jax.ShapeDtypeStruct((_N * 8 * _BT * 8 * 128,), jnp.float32),
    mesh=plsc.VectorSubcoreMesh(core_axis_name="c", subcore_axis_name="s"),
    compiler_params=pltpu.CompilerParams(use_tc_tiling_on_sc=False,
                                         needs_layout_passes=False),
    scratch_types=[
        pltpu.VMEM((_NT, 8, 128), jnp.int32),
        pltpu.VMEM((_N, 128), jnp.float32),
        pltpu.VMEM((50, 128), jnp.float32),
        pltpu.VMEM((128, _HIDDEN), jnp.float32),
        pltpu.VMEM((128, _HIDDEN), jnp.float32),
        pltpu.VMEM((_CW,), jnp.float32),
        pltpu.VMEM((_CW,), jnp.float32),
        pltpu.VMEM((_CW // 128, 128), jnp.int32),
        pltpu.VMEM((_HIDDEN,), jnp.float32),
        pltpu.VMEM((_HIDDEN,), jnp.float32),
        pltpu.VMEM_SHARED((_NS * 3 * _CW,), jnp.float32),
        pltpu.SemaphoreType.DMA,
        pltpu.SemaphoreType.DMA,
        pltpu.SemaphoreType.DMA,
        pltpu.SemaphoreType.DMA,
        pltpu.SemaphoreType.DMA,
        pltpu.SemaphoreType.DMA,
        pltpu.SemaphoreType.DMA,
    ],
)(_sc_body)


def kernel(x, in_degree, out_degree, atom_table, W, b):
    x4 = x.astype(jnp.int32).reshape(_BT, 128, _NT, 8).transpose(2, 0, 3, 1)
    din = in_degree.reshape(_BT, 128, _N).transpose(2, 0, 1)
    dout = out_degree.reshape(_BT, 128, _N).transpose(2, 0, 1)
    wf = W.reshape(-1)
    outf = _graph_node_feature_sc(x4, din, dout, atom_table, wf, b)
    out5 = outf.reshape(_N, 8, _BT, 8, 128)
    return out5.transpose(2, 4, 0, 1, 3).reshape(_B, _N, _HIDDEN)
